# baseline (speedup 1.0000x reference)
import functools

import jax
import jax.numpy as jnp
from jax import lax
from jax.experimental import pallas as pl
from jax.experimental.pallas import tpu as pltpu
from jax.experimental.pallas import tpu_sc as plsc

R = 128
R_SC = 64
N = 32768
L = 16
NV = N // L
K = 1024
KS = K * (1.0 - 2.0 ** -12)
WPL = K // L
NC = 2
NS = 16
NW = NC * NS
RPW = R_SC // NW
ACC = 4
NBUF = 3
SHW = 6

_mesh = plsc.VectorSubcoreMesh(
    core_axis_name="c", subcore_axis_name="s", num_cores=NC, num_subcores=NS
)


@functools.partial(
    pl.kernel,
    out_type=jax.ShapeDtypeStruct((R_SC, N), jnp.float32),
    mesh=_mesh,
    compiler_params=pltpu.CompilerParams(needs_layout_passes=False),
    scratch_types=[
        pltpu.VMEM((N,), jnp.float32),
        pltpu.VMEM((N,), jnp.float32),
        pltpu.VMEM((N,), jnp.float32),
        pltpu.VMEM((K,), jnp.float32),
        pltpu.VMEM((K,), jnp.float32),
        pltpu.SemaphoreType.DMA,
        pltpu.SemaphoreType.DMA,
        pltpu.SemaphoreType.DMA,
        pltpu.SemaphoreType.DMA,
        pltpu.SemaphoreType.DMA,
        pltpu.SemaphoreType.DMA,
    ],
)
def _sparsemax_sc(x_hbm, out_hbm, b0, b1, b2, cnt_v, sum_v,
                  is0, is1, is2, os0, os1, os2):
    wid = lax.axis_index("s") * NC + lax.axis_index("c")
    bufs = (b0, b1, b2)
    isems = (is0, is1, is2)
    osems = (os0, os1, os2)

    zeros = jnp.zeros((L,), jnp.float32)
    ones = jnp.ones((L,), jnp.float32)
    lane_i = lax.iota(jnp.int32, L)
    lane_f = lane_i.astype(jnp.float32)
    inv_k = jnp.float32(1.0 / KS)

    @plsc.parallel_loop(0, WPL, unroll=4)
    def _(w):
        cnt_v[pl.ds(w * L, L)] = zeros
        sum_v[pl.ds(w * L, L)] = zeros

    row0 = wid * RPW
    in_h = {}
    out_h = {}
    for j in range(min(NBUF, RPW)):
        in_h[j] = pltpu.async_copy(x_hbm.at[row0 + j], bufs[j], isems[j])

    in_h[0].wait()

    @plsc.parallel_loop(0, NV, ACC, unroll=2,
                        carry=tuple(jnp.full((L,), -jnp.inf, jnp.float32)
                                    for _ in range(ACC)))
    def m_accs0(i, accs):
        return tuple(
            jnp.maximum(a, b0[pl.ds((i + u) * L, L)])
            for u, a in enumerate(accs)
        )
    m_vec = functools.reduce(jnp.maximum, m_accs0)

    for j in range(RPW):
        b = j % NBUF
        row_v = bufs[b]
        if j >= NBUF - 1 and j + 1 < RPW:
            out_h[j + 1 - NBUF].wait()
            in_h[j + 1] = pltpu.async_copy(
                x_hbm.at[row0 + j + 1], bufs[(j + 1) % NBUF], isems[(j + 1) % NBUF]
            )

        m_s = jnp.max(m_vec)
        lo_s = m_s - jnp.float32(1.0)
        lo_v = jnp.full((L,), lo_s, jnp.float32)

        kf_v = jnp.full((L,), jnp.float32(KS), jnp.float32)

        @plsc.parallel_loop(0, NV, unroll=8)
        def _(i):
            x = row_v[pl.ds(i * L, L)]
            t = (x - lo_v) * kf_v
            idx = t.astype(jnp.int32)
            sidx = ((idx << 4) | (idx >> SHW)) & (K - 1)
            m = x >= lo_v
            plsc.addupdate_scatter(cnt_v, [sidx], ones, mask=m)
            plsc.addupdate_scatter(sum_v, [sidx], x, mask=m)

        lane_base = lo_v + lane_f * jnp.float32(WPL / KS)

        @plsc.parallel_loop(0, WPL, unroll=8, carry=(zeros, zeros))
        def tot(w, carry):
            c_run, s_run = carry
            sl = pl.ds(w * L, L)
            return c_run + cnt_v[sl], s_run + sum_v[sl]
        c_tot, s_tot = tot

        tc_excl = lax.rev(plsc.cumsum(lax.rev(c_tot, (0,))), (0,)) - c_tot
        ts_excl = lax.rev(plsc.cumsum(lax.rev(s_tot, (0,))), (0,)) - s_tot

        @plsc.parallel_loop(0, WPL, unroll=8,
                            carry=(tc_excl, ts_excl,
                                   jnp.full((L,), -3.0, jnp.float32)))
        def scan2(wr, carry):
            c_run, s_run, acc = carry
            w = WPL - 1 - wr
            sl = pl.ds(w * L, L)
            cs = c_run + cnt_v[sl]
            ss = s_run + sum_v[sl]
            edge = lane_base + w * inv_k
            g = ss - edge * cs - jnp.float32(1.0)
            r = (ss - jnp.float32(1.0)) / jnp.maximum(cs, jnp.float32(1e-30))
            cnt_v[sl] = zeros
            sum_v[sl] = zeros
            return cs, ss, jnp.maximum(acc, jnp.where(g >= 0, r, jnp.float32(-3.0)))
        rmax = scan2[2]
        tau1_v = jnp.full(
            (L,), jnp.clip(jnp.max(rmax), lo_s, m_s - jnp.float32(1.0 / N)),
            jnp.float32)

        if j + 1 < RPW:
            nxt_v = bufs[(j + 1) % NBUF]
            in_h[j + 1].wait()

            @plsc.parallel_loop(0, NV, ACC, unroll=2,
                                carry=tuple(jnp.full((L,), -jnp.inf, jnp.float32)
                                            for _ in range(ACC)))
            def cm_accs(i, accs):
                outs = []
                for u, a in enumerate(accs):
                    sl = pl.ds((i + u) * L, L)
                    row_v[sl] = jnp.maximum(row_v[sl] - tau1_v, zeros)
                    outs.append(jnp.maximum(a, nxt_v[sl]))
                return tuple(outs)
            m_vec = functools.reduce(jnp.maximum, cm_accs)
        else:
            @plsc.parallel_loop(0, NV, unroll=8)
            def _(i):
                sl = pl.ds(i * L, L)
                row_v[sl] = jnp.maximum(row_v[sl] - tau1_v, zeros)

        out_h[j] = pltpu.async_copy(row_v, out_hbm.at[row0 + j], osems[b])

    waited = {j + 1 - NBUF for j in range(RPW) if j >= NBUF - 1 and j + 1 < RPW}
    for j in range(RPW):
        if j not in waited:
            out_h[j].wait()


BR = 8
TC_ITERS = 18


def _tc_body(x_ref, o_ref):
    x = x_ref[...]
    m = jnp.max(x, axis=1, keepdims=True)
    lo = m - 1.0
    hi = m - jnp.float32(1.0 / N)

    def it(_, lh):
        lo, hi = lh
        t = 0.5 * (lo + hi)
        s = jnp.sum(jnp.maximum(x - t, 0.0), axis=1, keepdims=True)
        gt = s > 1.0
        return jnp.where(gt, t, lo), jnp.where(gt, hi, t)

    lo, hi = lax.fori_loop(0, TC_ITERS, it, (lo, hi))
    d = x - lo
    sp = jnp.sum(jnp.maximum(d, 0.0), axis=1, keepdims=True)
    kp = jnp.maximum(jnp.sum((d > 0.0).astype(jnp.float32), axis=1, keepdims=True), 1.0)
    tau = lo + (sp - 1.0) / kp
    o_ref[...] = jnp.maximum(x - tau, 0.0)


_sparsemax_tc = pl.pallas_call(
    _tc_body,
    grid=((R - R_SC) // BR,),
    in_specs=[pl.BlockSpec((BR, N), lambda i: (i + R_SC // BR, 0))],
    out_specs=pl.BlockSpec((BR, N), lambda i: (i, 0)),
    out_shape=jax.ShapeDtypeStruct((R - R_SC, N), jnp.float32),
)


def kernel(input):
    sc_out = _sparsemax_sc(input)
    tc_out = _sparsemax_tc(input)
    return jnp.concatenate([sc_out, tc_out], axis=0)


# speedup vs baseline: 1.5778x; 1.5778x over previous
import functools

import jax
import jax.numpy as jnp
from jax import lax
from jax.experimental import pallas as pl
from jax.experimental.pallas import tpu as pltpu
from jax.experimental.pallas import tpu_sc as plsc

R = 128
N = 32768
L = 16
NV = N // L
K = 1024
KS = K * (1.0 - 2.0 ** -12)
WPL = K // L
NC = 2
NS = 16
NW = NC * NS
RPW = R // NW
ACC = 4
NBUF = 3
SHW = 6

_mesh = plsc.VectorSubcoreMesh(
    core_axis_name="c", subcore_axis_name="s", num_cores=NC, num_subcores=NS
)


@functools.partial(
    pl.kernel,
    out_type=jax.ShapeDtypeStruct((R, N), jnp.float32),
    mesh=_mesh,
    compiler_params=pltpu.CompilerParams(needs_layout_passes=False),
    scratch_types=[
        pltpu.VMEM((N,), jnp.float32),
        pltpu.VMEM((N,), jnp.float32),
        pltpu.VMEM((N,), jnp.float32),
        pltpu.VMEM((K,), jnp.float32),
        pltpu.VMEM((K,), jnp.float32),
        pltpu.SemaphoreType.DMA,
        pltpu.SemaphoreType.DMA,
        pltpu.SemaphoreType.DMA,
        pltpu.SemaphoreType.DMA,
        pltpu.SemaphoreType.DMA,
        pltpu.SemaphoreType.DMA,
    ],
)
def _sparsemax_sc(x_hbm, out_hbm, b0, b1, b2, cnt_v, sum_v,
                  is0, is1, is2, os0, os1, os2):
    wid = lax.axis_index("s") * NC + lax.axis_index("c")
    bufs = (b0, b1, b2)
    isems = (is0, is1, is2)
    osems = (os0, os1, os2)

    zeros = jnp.zeros((L,), jnp.float32)
    ones = jnp.ones((L,), jnp.float32)
    lane_i = lax.iota(jnp.int32, L)
    lane_f = lane_i.astype(jnp.float32)
    inv_k = jnp.float32(1.0 / KS)

    @plsc.parallel_loop(0, WPL, unroll=4)
    def _(w):
        cnt_v[pl.ds(w * L, L)] = zeros
        sum_v[pl.ds(w * L, L)] = zeros

    row0 = wid * RPW
    in_h = {}
    out_h = {}
    for j in range(min(NBUF, RPW)):
        in_h[j] = pltpu.async_copy(x_hbm.at[row0 + j], bufs[j], isems[j])

    in_h[0].wait()

    @plsc.parallel_loop(0, NV, ACC, unroll=2,
                        carry=tuple(jnp.full((L,), -jnp.inf, jnp.float32)
                                    for _ in range(ACC)))
    def m_accs0(i, accs):
        return tuple(
            jnp.maximum(a, b0[pl.ds((i + u) * L, L)])
            for u, a in enumerate(accs)
        )
    m_vec = functools.reduce(jnp.maximum, m_accs0)

    for j in range(RPW):
        b = j % NBUF
        row_v = bufs[b]
        if j >= NBUF - 1 and j + 1 < RPW:
            out_h[j + 1 - NBUF].wait()
            in_h[j + 1] = pltpu.async_copy(
                x_hbm.at[row0 + j + 1], bufs[(j + 1) % NBUF], isems[(j + 1) % NBUF]
            )

        m_s = jnp.max(m_vec)
        lo_s = m_s - jnp.float32(1.0)
        lo_v = jnp.full((L,), lo_s, jnp.float32)

        kf_v = jnp.full((L,), jnp.float32(KS), jnp.float32)

        @plsc.parallel_loop(0, NV, unroll=16)
        def _(i):
            x = row_v[pl.ds(i * L, L)]
            t = (x - lo_v) * kf_v
            idx = t.astype(jnp.int32)
            sidx = ((idx << 4) | (idx >> SHW)) & (K - 1)
            m = x >= lo_v
            plsc.addupdate_scatter(cnt_v, [sidx], ones, mask=m)
            plsc.addupdate_scatter(sum_v, [sidx], x, mask=m)

        lane_base = lo_v + lane_f * jnp.float32(WPL / KS)

        @plsc.parallel_loop(0, WPL, unroll=8, carry=(zeros, zeros))
        def tot(w, carry):
            c_run, s_run = carry
            sl = pl.ds(w * L, L)
            return c_run + cnt_v[sl], s_run + sum_v[sl]
        c_tot, s_tot = tot

        tc_excl = lax.rev(plsc.cumsum(lax.rev(c_tot, (0,))), (0,)) - c_tot
        ts_excl = lax.rev(plsc.cumsum(lax.rev(s_tot, (0,))), (0,)) - s_tot

        @plsc.parallel_loop(0, WPL, unroll=8,
                            carry=(tc_excl, ts_excl,
                                   jnp.full((L,), -3.0, jnp.float32)))
        def scan2(wr, carry):
            c_run, s_run, acc = carry
            w = WPL - 1 - wr
            sl = pl.ds(w * L, L)
            cs = c_run + cnt_v[sl]
            ss = s_run + sum_v[sl]
            edge = lane_base + w * inv_k
            g = ss - edge * cs - jnp.float32(1.0)
            r = (ss - jnp.float32(1.0)) / jnp.maximum(cs, jnp.float32(1e-30))
            cnt_v[sl] = zeros
            sum_v[sl] = zeros
            return cs, ss, jnp.maximum(acc, jnp.where(g >= 0, r, jnp.float32(-3.0)))
        rmax = scan2[2]
        tau1_v = jnp.full(
            (L,), jnp.clip(jnp.max(rmax), lo_s, m_s - jnp.float32(1.0 / N)),
            jnp.float32)

        if j + 1 < RPW:
            nxt_v = bufs[(j + 1) % NBUF]
            in_h[j + 1].wait()

            @plsc.parallel_loop(0, NV, ACC, unroll=2,
                                carry=tuple(jnp.full((L,), -jnp.inf, jnp.float32)
                                            for _ in range(ACC)))
            def cm_accs(i, accs):
                outs = []
                for u, a in enumerate(accs):
                    sl = pl.ds((i + u) * L, L)
                    row_v[sl] = jnp.maximum(row_v[sl] - tau1_v, zeros)
                    outs.append(jnp.maximum(a, nxt_v[sl]))
                return tuple(outs)
            m_vec = functools.reduce(jnp.maximum, cm_accs)
        else:
            @plsc.parallel_loop(0, NV, unroll=8)
            def _(i):
                sl = pl.ds(i * L, L)
                row_v[sl] = jnp.maximum(row_v[sl] - tau1_v, zeros)

        out_h[j] = pltpu.async_copy(row_v, out_hbm.at[row0 + j], osems[b])

    waited = {j + 1 - NBUF for j in range(RPW) if j >= NBUF - 1 and j + 1 < RPW}
    for j in range(RPW):
        if j not in waited:
            out_h[j].wait()


def kernel(input):
    return _sparsemax_sc(input)


# speedup vs baseline: 1.5812x; 1.0022x over previous
import functools

import jax
import jax.numpy as jnp
from jax import lax
from jax.experimental import pallas as pl
from jax.experimental.pallas import tpu as pltpu
from jax.experimental.pallas import tpu_sc as plsc

R = 128
N = 32768
L = 16
NV = N // L
K = 1024
KS = K * (1.0 - 2.0 ** -12)
WPL = K // L
NC = 2
NS = 16
NW = NC * NS
RPW = R // NW
ACC = 4
NBUF = 3
SHW = 6

_mesh = plsc.VectorSubcoreMesh(
    core_axis_name="c", subcore_axis_name="s", num_cores=NC, num_subcores=NS
)


@functools.partial(
    pl.kernel,
    out_type=jax.ShapeDtypeStruct((R, N), jnp.float32),
    mesh=_mesh,
    compiler_params=pltpu.CompilerParams(
        needs_layout_passes=False,
        disable_bounds_checks=True,
        skip_device_barrier=True,
    ),
    scratch_types=[
        pltpu.VMEM((N,), jnp.float32),
        pltpu.VMEM((N,), jnp.float32),
        pltpu.VMEM((N,), jnp.float32),
        pltpu.VMEM((K,), jnp.float32),
        pltpu.VMEM((K,), jnp.float32),
        pltpu.SemaphoreType.DMA,
        pltpu.SemaphoreType.DMA,
        pltpu.SemaphoreType.DMA,
        pltpu.SemaphoreType.DMA,
        pltpu.SemaphoreType.DMA,
        pltpu.SemaphoreType.DMA,
    ],
)
def _sparsemax_sc(x_hbm, out_hbm, b0, b1, b2, cnt_v, sum_v,
                  is0, is1, is2, os0, os1, os2):
    wid = lax.axis_index("s") * NC + lax.axis_index("c")
    bufs = (b0, b1, b2)
    isems = (is0, is1, is2)
    osems = (os0, os1, os2)

    zeros = jnp.zeros((L,), jnp.float32)
    ones = jnp.ones((L,), jnp.float32)
    lane_i = lax.iota(jnp.int32, L)
    lane_f = lane_i.astype(jnp.float32)
    inv_k = jnp.float32(1.0 / KS)

    @plsc.parallel_loop(0, WPL, unroll=4)
    def _(w):
        cnt_v[pl.ds(w * L, L)] = zeros
        sum_v[pl.ds(w * L, L)] = zeros

    row0 = wid * RPW
    in_h = {}
    out_h = {}
    for j in range(min(NBUF, RPW)):
        in_h[j] = pltpu.async_copy(x_hbm.at[row0 + j], bufs[j], isems[j])

    in_h[0].wait()

    @plsc.parallel_loop(0, NV, ACC, unroll=2,
                        carry=tuple(jnp.full((L,), -jnp.inf, jnp.float32)
                                    for _ in range(ACC)))
    def m_accs0(i, accs):
        return tuple(
            jnp.maximum(a, b0[pl.ds((i + u) * L, L)])
            for u, a in enumerate(accs)
        )
    m_vec = functools.reduce(jnp.maximum, m_accs0)

    for j in range(RPW):
        b = j % NBUF
        row_v = bufs[b]
        if j >= NBUF - 1 and j + 1 < RPW:
            out_h[j + 1 - NBUF].wait()
            in_h[j + 1] = pltpu.async_copy(
                x_hbm.at[row0 + j + 1], bufs[(j + 1) % NBUF], isems[(j + 1) % NBUF]
            )

        m_s = jnp.max(m_vec)
        lo_s = m_s - jnp.float32(1.0)
        lo_v = jnp.full((L,), lo_s, jnp.float32)

        kf_v = jnp.full((L,), jnp.float32(KS), jnp.float32)

        @plsc.parallel_loop(0, NV, unroll=16)
        def _(i):
            x = row_v[pl.ds(i * L, L)]
            t = (x - lo_v) * kf_v
            idx = t.astype(jnp.int32)
            sidx = ((idx << 4) | (idx >> SHW)) & (K - 1)
            m = x >= lo_v
            plsc.addupdate_scatter(cnt_v, [sidx], ones, mask=m)
            plsc.addupdate_scatter(sum_v, [sidx], x, mask=m)

        lane_base = lo_v + lane_f * jnp.float32(WPL / KS)

        @plsc.parallel_loop(0, WPL, unroll=8, carry=(zeros, zeros))
        def tot(w, carry):
            c_run, s_run = carry
            sl = pl.ds(w * L, L)
            return c_run + cnt_v[sl], s_run + sum_v[sl]
        c_tot, s_tot = tot

        tc_excl = lax.rev(plsc.cumsum(lax.rev(c_tot, (0,))), (0,)) - c_tot
        ts_excl = lax.rev(plsc.cumsum(lax.rev(s_tot, (0,))), (0,)) - s_tot

        @plsc.parallel_loop(0, WPL, unroll=8,
                            carry=(tc_excl, ts_excl,
                                   jnp.full((L,), -3.0, jnp.float32)))
        def scan2(wr, carry):
            c_run, s_run, acc = carry
            w = WPL - 1 - wr
            sl = pl.ds(w * L, L)
            cs = c_run + cnt_v[sl]
            ss = s_run + sum_v[sl]
            edge = lane_base + w * inv_k
            g = ss - edge * cs - jnp.float32(1.0)
            r = (ss - jnp.float32(1.0)) / jnp.maximum(cs, jnp.float32(1e-30))
            cnt_v[sl] = zeros
            sum_v[sl] = zeros
            return cs, ss, jnp.maximum(acc, jnp.where(g >= 0, r, jnp.float32(-3.0)))
        rmax = scan2[2]
        tau1_v = jnp.full(
            (L,), jnp.clip(jnp.max(rmax), lo_s, m_s - jnp.float32(1.0 / N)),
            jnp.float32)

        if j + 1 < RPW:
            nxt_v = bufs[(j + 1) % NBUF]
            in_h[j + 1].wait()

            @plsc.parallel_loop(0, NV, ACC, unroll=2,
                                carry=tuple(jnp.full((L,), -jnp.inf, jnp.float32)
                                            for _ in range(ACC)))
            def cm_accs(i, accs):
                outs = []
                for u, a in enumerate(accs):
                    sl = pl.ds((i + u) * L, L)
                    row_v[sl] = jnp.maximum(row_v[sl] - tau1_v, zeros)
                    outs.append(jnp.maximum(a, nxt_v[sl]))
                return tuple(outs)
            m_vec = functools.reduce(jnp.maximum, cm_accs)
        else:
            @plsc.parallel_loop(0, NV, unroll=8)
            def _(i):
                sl = pl.ds(i * L, L)
                row_v[sl] = jnp.maximum(row_v[sl] - tau1_v, zeros)

        out_h[j] = pltpu.async_copy(row_v, out_hbm.at[row0 + j], osems[b])

    waited = {j + 1 - NBUF for j in range(RPW) if j >= NBUF - 1 and j + 1 < RPW}
    for j in range(RPW):
        if j not in waited:
            out_h[j].wait()


def kernel(input):
    return _sparsemax_sc(input)
